# Initial kernel scaffold; baseline (speedup 1.0000x reference)
import jax
import jax.numpy as jnp
from jax.experimental import pallas as pl


def kernel(loc_data, conf_data, loc_data_vis, conf_data_vis, fcn_output, fcn_visible_output, priors, targets, targets_vis, seg_targets, seg_visible_targets):
    raise NotImplementedError("write your pallas kernel here")


# speedup vs baseline: 12.8327x; 12.8327x over previous
import jax
import jax.numpy as jnp
from jax.experimental import pallas as pl

THRESHOLD = 0.5
NEGPOS_RATIO = 3
V0, V1 = 0.1, 0.2


def _match_body(tgt_ref, tgtv_ref, priors_ref,
                loct_ref, ct_ref, loctv_ref, ctv_ref, hp_ref):
    i = pl.program_id(0)
    pt = priors_ref[...]
    Pn = pt.shape[1]
    pcx, pcy = pt[0:1, :], pt[1:2, :]
    pw, ph = pt[2:3, :], pt[3:4, :]
    px0, py0 = pcx - pw * 0.5, pcy - ph * 0.5
    px1, py1 = pcx + pw * 0.5, pcy + ph * 0.5
    area_b = pw * ph

    def match(tref):
        tg = tref[0]
        T = tg.shape[0]
        tx0, ty0 = tg[:, 0:1], tg[:, 1:2]
        tx1, ty1 = tg[:, 2:3], tg[:, 3:4]
        lab = tg[:, 4:5]
        ix = jnp.maximum(jnp.minimum(px1, tx1) - jnp.maximum(px0, tx0), 0.0)
        iy = jnp.maximum(jnp.minimum(py1, ty1) - jnp.maximum(py0, ty0), 0.0)
        inter = ix * iy
        area_a = (tx1 - tx0) * (ty1 - ty0)
        iou = inter / (area_a + area_b - inter)
        it = jax.lax.broadcasted_iota(jnp.int32, iou.shape, 0)
        ip = jax.lax.broadcasted_iota(jnp.int32, iou.shape, 1)
        bto = jnp.max(iou, axis=0, keepdims=True)
        bti = jnp.min(jnp.where(iou == bto, it, T), axis=0, keepdims=True)
        bpo = jnp.max(iou, axis=1, keepdims=True)
        bpi = jnp.min(jnp.where(iou == bpo, ip, Pn), axis=1, keepdims=True)
        ov = ip == bpi
        any_ov = jnp.max(ov.astype(jnp.int32), axis=0, keepdims=True) > 0
        ov_t = jnp.max(jnp.where(ov, it, -1), axis=0, keepdims=True)
        bto = jnp.where(any_ov, 2.0, bto)
        bti = jnp.where(any_ov, ov_t, bti)
        onehot = (it == bti).astype(jnp.float32)
        mx0 = jnp.sum(onehot * tx0, axis=0, keepdims=True)
        my0 = jnp.sum(onehot * ty0, axis=0, keepdims=True)
        mx1 = jnp.sum(onehot * tx1, axis=0, keepdims=True)
        my1 = jnp.sum(onehot * ty1, axis=0, keepdims=True)
        mlab = jnp.sum(onehot * lab, axis=0, keepdims=True)
        conf = mlab.astype(jnp.int32) + 1
        conf = jnp.where(bto < THRESHOLD, 0, conf)
        gcx = ((mx0 + mx1) * 0.5 - pcx) / (V0 * pw)
        gcy = ((my0 + my1) * 0.5 - pcy) / (V0 * ph)
        gw = jnp.log((mx1 - mx0) / pw) / V1
        gh = jnp.log((my1 - my0) / ph) / V1
        loc = jnp.concatenate([gcx, gcy, gw, gh], axis=0)
        return loc, conf

    loc, conf = match(tgt_ref)
    locv, confv = match(tgtv_ref)
    loct_ref[0] = loc
    ct_ref[0] = conf
    loctv_ref[0] = locv
    ctv_ref[0] = confv
    pos_any = jnp.max(((conf > 0) & (confv > 0)).astype(jnp.float32),
                      axis=1, keepdims=True)

    @pl.when(i == 0)
    def _():
        hp_ref[...] = jnp.zeros((1, 1), jnp.float32)

    hp_ref[...] = jnp.maximum(hp_ref[...], pos_any)


def _loss_b1_body(conf_ref, confv_ref, loc_ref, locv_ref,
                  loct_ref, loctv_ref, ct_ref, ctv_ref, hp_ref,
                  v_ref, cevn_ref, npos_ref, ll_ref, llv_ref,
                  lcp_ref, lcvp_ref):
    j = pl.program_id(1)
    cf = conf_ref[0]
    cfv = confv_ref[0]
    ct = ct_ref[0]
    ctv = ctv_ref[0]
    hp_f = hp_ref[...]
    pa0_f = (ct > 0).astype(jnp.float32)
    pv0_f = (ctv > 0).astype(jnp.float32)
    pb_f = pa0_f * pv0_f
    pa_f = hp_f * pb_f + (1.0 - hp_f) * pa0_f
    pv_f = hp_f * pb_f + (1.0 - hp_f) * pv0_f

    ic = jax.lax.broadcasted_iota(jnp.int32, cf.shape, 0)
    m = jnp.max(cf, axis=0, keepdims=True)
    lse = m + jnp.log(jnp.sum(jnp.exp(cf - m), axis=0, keepdims=True))
    gath = jnp.sum(jnp.where(ic == ct, cf, 0.0), axis=0, keepdims=True)
    ce = lse - gath
    mv = jnp.max(cfv, axis=0, keepdims=True)
    lsev = mv + jnp.log(jnp.sum(jnp.exp(cfv - mv), axis=0, keepdims=True))
    gathv = jnp.sum(jnp.where(ic == ctv, cfv, 0.0), axis=0, keepdims=True)
    cev = lsev - gathv

    d = loc_ref[0] - loct_ref[0]
    ad = jnp.abs(d)
    s1 = jnp.where(ad < 1.0, 0.5 * d * d, ad - 0.5)
    ll = jnp.sum(s1 * pa_f)
    dv = locv_ref[0] - loctv_ref[0]
    adv = jnp.abs(dv)
    s1v = jnp.where(adv < 1.0, 0.5 * dv * dv, adv - 0.5)
    llv = jnp.sum(s1v * pv_f)

    v = jnp.maximum(ce * (1.0 - pa_f), 0.0)
    v_ref[0] = v
    cevn_ref[0] = cev * (1.0 - pv_f)

    num_pos = jnp.sum(pa_f)
    lcp = jnp.sum(ce * pa_f)
    lcvp = jnp.sum(cev * pv_f)

    @pl.when(j == 0)
    def _():
        z = jnp.zeros((1, 1, 1), jnp.float32)
        npos_ref[...] = z
        ll_ref[...] = z
        llv_ref[...] = z
        lcp_ref[...] = z
        lcvp_ref[...] = z

    npos_ref[...] += jnp.full((1, 1, 1), num_pos)
    ll_ref[...] += jnp.full((1, 1, 1), ll)
    llv_ref[...] += jnp.full((1, 1, 1), llv)
    lcp_ref[...] += jnp.full((1, 1, 1), lcp)
    lcvp_ref[...] += jnp.full((1, 1, 1), lcvp)


def _loss_b2_body(v_ref, cevn_ref, npos_ref, topk_ref, visx_ref):
    v = v_ref[0]
    Pn = v.shape[1]
    cevn = cevn_ref[0]
    num_pos = jnp.sum(npos_ref[...])
    k = jnp.minimum(NEGPOS_RATIO * num_pos.astype(jnp.int32),
                    jnp.int32(Pn - 1))
    bits = jax.lax.bitcast_convert_type(v, jnp.int32)

    def bs_body(_, c):
        lo, hi = c
        mid = lo + (hi - lo) // 2
        cnt = jnp.sum((bits > mid).astype(jnp.int32))
        big = cnt >= k
        return (jnp.where(big, mid + 1, lo), jnp.where(big, hi, mid))

    lo, _ = jax.lax.fori_loop(0, 31, bs_body,
                              (jnp.int32(0), jnp.int32(0x7F800000)))
    t = jax.lax.bitcast_convert_type(lo, jnp.float32)
    gt = bits > lo
    cnt_gt = jnp.sum(gt.astype(jnp.int32))
    r = k - cnt_gt
    topk = jnp.sum(jnp.where(gt, v, 0.0)) + r.astype(jnp.float32) * t

    eq = bits == lo
    ipx = jax.lax.broadcasted_iota(jnp.int32, eq.shape, 1)

    def bs2_body(_, c):
        lo2, hi2 = c
        mid = lo2 + (hi2 - lo2) // 2
        cnt = jnp.sum((eq & (ipx < mid)).astype(jnp.int32))
        enough = cnt >= r
        return (jnp.where(enough, lo2, mid + 1), jnp.where(enough, mid, hi2))

    _, mcut = jax.lax.fori_loop(0, 16, bs2_body,
                                (jnp.int32(0), jnp.int32(Pn)))
    neg = gt | (eq & (ipx < mcut))
    visx = jnp.sum(jnp.where(neg, cevn, 0.0))

    topk_ref[...] = jnp.full((1, 1, 1), topk)
    visx_ref[...] = jnp.full((1, 1, 1), visx)


def _seg_body(f_ref, fv_ref, t_ref, tv_ref, s_ref, sv_ref):
    i = pl.program_id(0)
    j = pl.program_id(1)

    @pl.when((i == 0) & (j == 0))
    def _():
        s_ref[...] = jnp.zeros((1, 1), jnp.float32)
        sv_ref[...] = jnp.zeros((1, 1), jnp.float32)

    def nll_sum(x, tg):
        ic = jax.lax.broadcasted_iota(jnp.int32, x.shape, 0)
        m = jnp.max(x, axis=0, keepdims=True)
        lse = m + jnp.log(jnp.sum(jnp.exp(x - m), axis=0, keepdims=True))
        g = jnp.sum(jnp.where(ic == tg, x, 0.0), axis=0, keepdims=True)
        return jnp.sum(lse - g, axis=1, keepdims=True)

    s_ref[...] = s_ref[...] + nll_sum(f_ref[0], t_ref[0])
    sv_ref[...] = sv_ref[...] + nll_sum(fv_ref[0], tv_ref[0])


def kernel(loc_data, conf_data, loc_data_vis, conf_data_vis, fcn_output,
           fcn_visible_output, priors, targets, targets_vis, seg_targets,
           seg_visible_targets):
    B, Pn, C = conf_data.shape
    T = targets.shape[1]
    _, Cs, H, W = fcn_output.shape
    HW = H * W
    f32 = jnp.float32

    priors_t = jnp.transpose(priors, (1, 0))
    conf_t8 = jnp.transpose(conf_data, (0, 2, 1))
    confv_t8 = jnp.transpose(conf_data_vis, (0, 2, 1))
    loc_t8 = jnp.transpose(loc_data, (0, 2, 1))
    locv_t8 = jnp.transpose(loc_data_vis, (0, 2, 1))

    loct, ct, loctv, ctv, haspos = pl.pallas_call(
        _match_body,
        grid=(B,),
        in_specs=[
            pl.BlockSpec((1, T, 5), lambda i: (i, 0, 0)),
            pl.BlockSpec((1, T, 5), lambda i: (i, 0, 0)),
            pl.BlockSpec((4, Pn), lambda i: (0, 0)),
        ],
        out_specs=[
            pl.BlockSpec((1, 4, Pn), lambda i: (i, 0, 0)),
            pl.BlockSpec((1, 1, Pn), lambda i: (i, 0, 0)),
            pl.BlockSpec((1, 4, Pn), lambda i: (i, 0, 0)),
            pl.BlockSpec((1, 1, Pn), lambda i: (i, 0, 0)),
            pl.BlockSpec((1, 1), lambda i: (0, 0)),
        ],
        out_shape=[
            jax.ShapeDtypeStruct((B, 4, Pn), f32),
            jax.ShapeDtypeStruct((B, 1, Pn), jnp.int32),
            jax.ShapeDtypeStruct((B, 4, Pn), f32),
            jax.ShapeDtypeStruct((B, 1, Pn), jnp.int32),
            jax.ShapeDtypeStruct((1, 1), f32),
        ],
    )(targets, targets_vis, priors_t)

    BP = 8192
    nB = Pn // BP
    v_arr, cevn, npos, ll, llv, lcp, lcvp = pl.pallas_call(
        _loss_b1_body,
        grid=(B, nB),
        in_specs=[
            pl.BlockSpec((1, C, BP), lambda i, j: (i, 0, j)),
            pl.BlockSpec((1, C, BP), lambda i, j: (i, 0, j)),
            pl.BlockSpec((1, 4, BP), lambda i, j: (i, 0, j)),
            pl.BlockSpec((1, 4, BP), lambda i, j: (i, 0, j)),
            pl.BlockSpec((1, 4, BP), lambda i, j: (i, 0, j)),
            pl.BlockSpec((1, 4, BP), lambda i, j: (i, 0, j)),
            pl.BlockSpec((1, 1, BP), lambda i, j: (i, 0, j)),
            pl.BlockSpec((1, 1, BP), lambda i, j: (i, 0, j)),
            pl.BlockSpec((1, 1), lambda i, j: (0, 0)),
        ],
        out_specs=[
            pl.BlockSpec((1, 1, BP), lambda i, j: (i, 0, j)),
            pl.BlockSpec((1, 1, BP), lambda i, j: (i, 0, j)),
        ] + [pl.BlockSpec((1, 1, 1), lambda i, j: (i, 0, 0))] * 5,
        out_shape=[
            jax.ShapeDtypeStruct((B, 1, Pn), f32),
            jax.ShapeDtypeStruct((B, 1, Pn), f32),
        ] + [jax.ShapeDtypeStruct((B, 1, 1), f32)] * 5,
    )(conf_t8, confv_t8, loc_t8, locv_t8, loct, loctv, ct, ctv, haspos)

    topk, visx = pl.pallas_call(
        _loss_b2_body,
        grid=(B,),
        in_specs=[
            pl.BlockSpec((1, 1, Pn), lambda i: (i, 0, 0)),
            pl.BlockSpec((1, 1, Pn), lambda i: (i, 0, 0)),
            pl.BlockSpec((1, 1, 1), lambda i: (i, 0, 0)),
        ],
        out_specs=[pl.BlockSpec((1, 1, 1), lambda i: (i, 0, 0))] * 2,
        out_shape=[jax.ShapeDtypeStruct((B, 1, 1), f32)] * 2,
    )(v_arr, cevn, npos)

    L = 19200
    nL = HW // L
    fcn2 = fcn_output.reshape(B, Cs, HW)
    fcnv2 = fcn_visible_output.reshape(B, Cs, HW)
    segt = seg_targets.reshape(B, 1, HW).astype(jnp.int32)
    segtv = seg_visible_targets.reshape(B, 1, HW).astype(jnp.int32)

    s, sv = pl.pallas_call(
        _seg_body,
        grid=(B, nL),
        in_specs=[
            pl.BlockSpec((1, Cs, L), lambda i, j: (i, 0, j)),
            pl.BlockSpec((1, Cs, L), lambda i, j: (i, 0, j)),
            pl.BlockSpec((1, 1, L), lambda i, j: (i, 0, j)),
            pl.BlockSpec((1, 1, L), lambda i, j: (i, 0, j)),
        ],
        out_specs=[
            pl.BlockSpec((1, 1), lambda i, j: (0, 0)),
            pl.BlockSpec((1, 1), lambda i, j: (0, 0)),
        ],
        out_shape=[
            jax.ShapeDtypeStruct((1, 1), f32),
            jax.ShapeDtypeStruct((1, 1), f32),
        ],
    )(fcn2, fcnv2, segt, segtv)

    N = jnp.sum(npos)
    denom = jnp.float32(B * HW)
    return (jnp.sum(ll) / N, (jnp.sum(lcp) + jnp.sum(topk)) / N,
            jnp.sum(llv) / N, (jnp.sum(lcvp) + jnp.sum(visx)) / N,
            s[0, 0] / denom, sv[0, 0] / denom)
